# baseline (speedup 1.0000x reference)
import functools

import jax
import jax.numpy as jnp
from jax import lax
from jax.experimental import pallas as pl
from jax.experimental.pallas import tpu as pltpu
from jax.experimental.pallas import tpu_sc as plsc

_LANES = 128
_PB = 4
_TC = 256
_TBLK = 4096


def _pad_body(x_ref, o_ref):
    S = x_ref.shape[1]
    z = jnp.zeros((x_ref.shape[0], _LANES - S), jnp.int32)
    o_ref[...] = jnp.concatenate([x_ref[...], z], axis=1)


def _pad_x(x):
    Bt, S = x.shape
    BLK = 2048
    return pl.pallas_call(
        _pad_body,
        grid=(Bt // BLK,),
        in_specs=[pl.BlockSpec((BLK, S), lambda i: (i, 0))],
        out_specs=pl.BlockSpec((BLK, _LANES), lambda i: (i, 0)),
        out_shape=jax.ShapeDtypeStruct((Bt, _LANES), jnp.int32),
    )(x)


def _tt_body(i_ref, at_ref, o_ref):
    D = at_ref.shape[0]
    ident = i_ref[...]
    z = jnp.zeros((_TC, _LANES - D), jnp.float32)
    for j in range(_TBLK // _TC):
        sl = at_ref[:, j * _TC:(j + 1) * _TC]
        tj = lax.dot_general(
            ident, sl, (((1,), (1,)), ((), ())),
            preferred_element_type=jnp.float32,
        )
        o_ref[pl.ds(j * _TC, _TC), pl.ds(0, D)] = tj
        o_ref[pl.ds(j * _TC, _TC), pl.ds(D, _LANES - D)] = z


def _tt_transpose(tableT):
    D, V = tableT.shape
    nblk = -(-V // _TBLK)
    M2 = nblk * _TBLK
    ident = jnp.eye(_TC, dtype=jnp.float32)
    return pl.pallas_call(
        _tt_body,
        grid=(nblk,),
        in_specs=[
            pl.BlockSpec((_TC, _TC), lambda i: (0, 0)),
            pl.BlockSpec((D, _TBLK), lambda i: (0, i)),
        ],
        out_specs=pl.BlockSpec((_TBLK, _LANES), lambda i: (i, 0)),
        out_shape=jax.ShapeDtypeStruct((M2, _LANES), jnp.float32),
    )(ident, tableT)


def _sc_gather(ttv, xpad, Bt, SP, D):
    info = plsc.get_sparse_core_info()
    NC, NS = info.num_cores, info.num_subcores
    NW = NC * NS
    assert Bt % NW == 0
    b_per_w = Bt // NW
    rows_w = b_per_w * SP
    XC = 64
    GC = 128
    SUPER = 512
    n_super = rows_w // SUPER
    n_g = SUPER // GC

    mesh = plsc.VectorSubcoreMesh(core_axis_name="c", subcore_axis_name="s")

    @functools.partial(
        pl.kernel,
        mesh=mesh,
        compiler_params=pltpu.CompilerParams(use_tc_tiling_on_sc=False),
        out_type=jax.ShapeDtypeStruct((Bt * SP, D), jnp.float32),
        scratch_types=[
            pltpu.VMEM((XC, 128), jnp.int32),
            pltpu.VMEM((rows_w // SP * SP,), jnp.int32),
            pltpu.VMEM((SUPER, D), jnp.float32),
            pltpu.SemaphoreType.DMA,
        ],
    )
    def k(tt_hbm, x_hbm, out_hbm, idx2d, idxc, rows_v, gsem):
        wid = lax.axis_index("s") * NC + lax.axis_index("c")
        wbase = wid * b_per_w

        def prep(c, carry):
            pltpu.sync_copy(x_hbm.at[pl.ds(wbase + c * XC, XC)], idx2d)

            def prow(r, carry2):
                for g in range(SP // 16):
                    v = idx2d[r, pl.ds(g * 16, 16)]
                    idxc[pl.ds((c * XC + r) * SP + g * 16, 16)] = v + v
                return carry2

            lax.fori_loop(0, XC, prow, 0)
            return carry

        lax.fori_loop(0, b_per_w // XC, prep, 0)

        def body(sc, carry):
            descs = []
            for j in range(n_g):
                d = pltpu.async_copy(
                    tt_hbm.at[idxc.at[pl.ds(sc * SUPER + j * GC, GC)]],
                    rows_v.at[pl.ds(j * GC, GC)],
                    gsem,
                )
                descs.append(d)
            for d in descs:
                d.wait()
            pltpu.sync_copy(
                rows_v,
                out_hbm.at[pl.ds(wid * rows_w + sc * SUPER, SUPER)],
            )
            return carry

        lax.fori_loop(0, n_super, body, 0)

    return k(ttv, xpad)


_SQRT_HALF = 0.7071067811865476


def _make_adapter_body(BB, S, D, SP):
    def body(x_ref, w_ref, b_ref, o_ref):
        v3 = x_ref[...]
        w2 = w_ref[...]
        bcol = b_ref[...]
        for j in range(S // 2):
            e2 = v3[:, j]
            h2 = lax.dot_general(
                w2, e2, (((1,), (1,)), ((), ())),
                preferred_element_type=jnp.float32,
            )
            h2 = h2 + bcol
            g = h2 * 0.5 * (1.0 + lax.erf(h2 * _SQRT_HALF))
            o_ref[2 * j] = g[:D]
            o_ref[2 * j + 1] = g[D:]

    return body


def _tc_adapter(G3, W2, bcol, Bt, S, D, SP):
    BB = 128
    assert Bt % BB == 0 and S % 2 == 0
    return pl.pallas_call(
        _make_adapter_body(BB, S, D, SP),
        grid=(Bt // BB,),
        in_specs=[
            pl.BlockSpec((BB, SP // 2, _LANES), lambda i: (i, 0, 0)),
            pl.BlockSpec((_LANES, _LANES), lambda i: (0, 0)),
            pl.BlockSpec((_LANES, 1), lambda i: (0, 0)),
        ],
        out_specs=pl.BlockSpec((S, D, BB), lambda i: (0, 0, i)),
        out_shape=jax.ShapeDtypeStruct((S, D, Bt), jnp.float32),
    )(G3, W2, bcol)


def kernel(x, table, W, b):
    Bt, S = x.shape
    V, D = table.shape
    SP = 64
    xpad = _pad_x(x.astype(jnp.int32))
    tt = _tt_transpose(table.T)
    ttv = tt.reshape(tt.shape[0] * 2, D)
    G = _sc_gather(ttv, xpad, Bt, SP, D)
    G3 = G.reshape(Bt, SP // 2, 2 * D)
    W2 = (
        jnp.zeros((_LANES, _LANES), jnp.float32)
        .at[:D, :D].set(W)
        .at[D:, D:].set(W)
    )
    bcol = jnp.concatenate([b, b]).reshape(_LANES, 1)
    outT = _tc_adapter(G3, W2, bcol, Bt, S, D, SP)
    return jnp.transpose(outT, (2, 0, 1))


# speedup vs baseline: 1.0022x; 1.0022x over previous
import functools

import jax
import jax.numpy as jnp
from jax import lax
from jax.experimental import pallas as pl
from jax.experimental.pallas import tpu as pltpu
from jax.experimental.pallas import tpu_sc as plsc

_LANES = 128
_PB = 4
_TC = 256
_TBLK = 4096


def _pad_body(x_ref, o_ref):
    S = x_ref.shape[1]
    z = jnp.zeros((x_ref.shape[0], _LANES - S), jnp.int32)
    v = x_ref[...]
    o_ref[...] = jnp.concatenate([v + v, z], axis=1)


def _pad_x(x):
    Bt, S = x.shape
    BLK = 2048
    return pl.pallas_call(
        _pad_body,
        grid=(Bt // BLK,),
        in_specs=[pl.BlockSpec((BLK, S), lambda i: (i, 0))],
        out_specs=pl.BlockSpec((BLK, _LANES), lambda i: (i, 0)),
        out_shape=jax.ShapeDtypeStruct((Bt, _LANES), jnp.int32),
    )(x)


def _tt_body(i_ref, at_ref, o_ref):
    D = at_ref.shape[0]
    ident = i_ref[...]
    z = jnp.zeros((_TC, _LANES - D), jnp.float32)
    for j in range(_TBLK // _TC):
        sl = at_ref[:, j * _TC:(j + 1) * _TC]
        tj = lax.dot_general(
            ident, sl, (((1,), (1,)), ((), ())),
            preferred_element_type=jnp.float32,
        )
        o_ref[pl.ds(j * _TC, _TC), pl.ds(0, D)] = tj
        o_ref[pl.ds(j * _TC, _TC), pl.ds(D, _LANES - D)] = z


def _tt_transpose(tableT):
    D, V = tableT.shape
    nblk = -(-V // _TBLK)
    M2 = nblk * _TBLK
    ident = jnp.eye(_TC, dtype=jnp.float32)
    return pl.pallas_call(
        _tt_body,
        grid=(nblk,),
        in_specs=[
            pl.BlockSpec((_TC, _TC), lambda i: (0, 0)),
            pl.BlockSpec((D, _TBLK), lambda i: (0, i)),
        ],
        out_specs=pl.BlockSpec((_TBLK, _LANES), lambda i: (i, 0)),
        out_shape=jax.ShapeDtypeStruct((M2, _LANES), jnp.float32),
    )(ident, tableT)


def _sc_gather(ttv, x2pad, Bt, SP, D):
    info = plsc.get_sparse_core_info()
    NC, NS = info.num_cores, info.num_subcores
    NW = NC * NS
    assert Bt % NW == 0
    b_per_w = Bt // NW
    PB = 8
    n_super = b_per_w // PB
    SUPER = PB * SP

    mesh = plsc.VectorSubcoreMesh(core_axis_name="c", subcore_axis_name="s")

    @functools.partial(
        pl.kernel,
        mesh=mesh,
        compiler_params=pltpu.CompilerParams(use_tc_tiling_on_sc=False),
        out_type=jax.ShapeDtypeStruct((Bt * SP, D), jnp.float32),
        scratch_types=[
            pltpu.VMEM((b_per_w, _LANES), jnp.int32),
            pltpu.VMEM((SUPER, D), jnp.float32),
            pltpu.SemaphoreType.DMA,
        ],
    )
    def k(tt_hbm, x_hbm, out_hbm, idx_v, rows_v, gsem):
        wid = lax.axis_index("s") * NC + lax.axis_index("c")
        wbase = wid * b_per_w
        pltpu.sync_copy(x_hbm.at[pl.ds(wbase, b_per_w)], idx_v)

        def body(sc, carry):
            descs = []
            for bb in range(PB):
                d = pltpu.async_copy(
                    tt_hbm.at[idx_v.at[sc * PB + bb, pl.ds(0, SP)]],
                    rows_v.at[pl.ds(bb * SP, SP)],
                    gsem,
                )
                descs.append(d)
            for d in descs:
                d.wait()
            pltpu.sync_copy(
                rows_v,
                out_hbm.at[pl.ds((wbase + sc * PB) * SP, SUPER)],
            )
            return carry

        lax.fori_loop(0, n_super, body, 0)

    return k(ttv, x2pad)


_SQRT_HALF = 0.7071067811865476


def _make_adapter_body(BB, S, D, SP):
    def body(x_ref, w_ref, b_ref, o_ref):
        v3 = x_ref[...]
        w2 = w_ref[...]
        bcol = b_ref[...]
        for j in range(S // 2):
            e2 = v3[:, j]
            h2 = lax.dot_general(
                w2, e2, (((1,), (1,)), ((), ())),
                preferred_element_type=jnp.float32,
            )
            h2 = h2 + bcol
            g = h2 * 0.5 * (1.0 + lax.erf(h2 * _SQRT_HALF))
            o_ref[2 * j] = g[:D]
            o_ref[2 * j + 1] = g[D:]

    return body


def _tc_adapter(G3, W2, bcol, Bt, S, D, SP):
    BB = 128
    assert Bt % BB == 0 and S % 2 == 0
    return pl.pallas_call(
        _make_adapter_body(BB, S, D, SP),
        grid=(Bt // BB,),
        in_specs=[
            pl.BlockSpec((BB, SP // 2, _LANES), lambda i: (i, 0, 0)),
            pl.BlockSpec((_LANES, _LANES), lambda i: (0, 0)),
            pl.BlockSpec((_LANES, 1), lambda i: (0, 0)),
        ],
        out_specs=pl.BlockSpec((S, D, BB), lambda i: (0, 0, i)),
        out_shape=jax.ShapeDtypeStruct((S, D, Bt), jnp.float32),
    )(G3, W2, bcol)


def kernel(x, table, W, b):
    Bt, S = x.shape
    V, D = table.shape
    SP = 64
    xpad = _pad_x(x.astype(jnp.int32))
    tt = _tt_transpose(table.T)
    ttv = tt.reshape(tt.shape[0] * 2, D)
    G = _sc_gather(ttv, xpad, Bt, SP, D)
    G3 = G.reshape(Bt, SP // 2, 2 * D)
    W2 = (
        jnp.zeros((_LANES, _LANES), jnp.float32)
        .at[:D, :D].set(W)
        .at[D:, D:].set(W)
    )
    bcol = jnp.concatenate([b, b]).reshape(_LANES, 1)
    outT = _tc_adapter(G3, W2, bcol, Bt, S, D, SP)
    return jnp.transpose(outT, (2, 0, 1))
